# Initial kernel scaffold; baseline (speedup 1.0000x reference)
import functools

import jax
import jax.numpy as jnp
from jax import lax
from jax.experimental import pallas as pl
from jax.experimental.pallas import tpu as pltpu
from jax.experimental.pallas import tpu_sc as plsc

N = 50000
E = 800000

RB = 512
N_PAD = 50176
G = N_PAD // RB

NC, NS = 2, 16
NW = NC * NS
CH = 128
E_PAD = 802816
EPW = E_PAD // NW
NCH = EPW // CH


def _elu(x):
    return jnp.where(x > 0, x, jnp.expm1(x))


def _enc_body(oh, ft, w0a, w0b, b0, w1, b1, wc0, ws0, b0c, m0, t0):
    h = oh[...] @ w0a[...] + ft[...] @ w0b[...] + b0[...]
    h = _elu(h)
    h = _elu(h @ w1[...] + b1[...])
    m0[...] = h @ wc0[...]
    t0[...] = h @ ws0[...] + b0c[...]


def _mid_body(p0, p1, t0, wc1, ws1, b1c, m1, t1):
    x1 = _elu(p0[...] + p1[...] + t0[...])
    m1[...] = x1 @ wc1[...]
    t1[...] = x1 @ ws1[...] + b1c[...]


def _score_body(q0, q1, t1, w0, b0, w1, b1, w2, b2, fw, fb, out):
    x2 = _elu(q0[...] + q1[...] + t1[...])
    h = _elu(x2 @ w0[...] + b0[...])
    h = _elu(h @ w1[...] + b1[...])
    h = _elu(h @ w2[...] + b2[...])
    out[...] = jax.nn.sigmoid(h @ fw[...] + fb[...])


def _row_spec(d):
    return pl.BlockSpec((RB, d), lambda i: (i, 0))


def _full_spec(shape):
    return pl.BlockSpec(shape, lambda i: tuple(0 for _ in shape))


def _tc_call(body, in_arrays, n_row_args, out_widths):
    in_specs = [_row_spec(a.shape[1]) for a in in_arrays[:n_row_args]]
    in_specs += [_full_spec(a.shape) for a in in_arrays[n_row_args:]]
    out_shape = [jax.ShapeDtypeStruct((N_PAD, d), jnp.float32) for d in out_widths]
    out_specs = [_row_spec(d) for d in out_widths]
    res = pl.pallas_call(
        body,
        grid=(G,),
        in_specs=in_specs,
        out_specs=out_specs,
        out_shape=out_shape,
        compiler_params=pltpu.CompilerParams(
            dimension_semantics=("arbitrary",)),
    )(*in_arrays)
    return res


def _make_sc_segsum(d):
    mesh = plsc.VectorSubcoreMesh(core_axis_name="c", subcore_axis_name="s")
    rpt = N_PAD // NS

    @functools.partial(
        pl.kernel,
        mesh=mesh,
        out_type=[jax.ShapeDtypeStruct((N_PAD, d), jnp.float32)] * 2,
        scratch_types=[
            pltpu.VMEM((NCH, CH), jnp.int32),
            pltpu.VMEM((NCH, CH), jnp.int32),
            pltpu.VMEM((CH, d), jnp.float32),
            pltpu.VMEM_SHARED((N_PAD, d), jnp.float32),
            pltpu.SemaphoreType.DMA,
        ],
    )
    def sck(table, src2d, dst2d, zeros, out0, out1, srcv, dstv, rows, acc, sem):
        cid = lax.axis_index("c")
        sid = lax.axis_index("s")
        wid = cid * NS + sid

        pltpu.sync_copy(zeros.at[pl.ds(sid * rpt, rpt)],
                        acc.at[pl.ds(sid * rpt, rpt)])
        pltpu.sync_copy(src2d.at[pl.ds(wid * NCH, NCH)], srcv)
        pltpu.sync_copy(dst2d.at[pl.ds(wid * NCH, NCH)], dstv)
        plsc.subcore_barrier()

        def step(j, carry):
            pltpu.async_copy(table.at[srcv.at[j]], rows, sem).wait()
            pltpu.sync_copy(rows, acc.at[dstv.at[j]], add=True)
            return carry

        lax.fori_loop(0, NCH, step, 0, unroll=False)

        plsc.subcore_barrier()
        sl = pl.ds(sid * rpt, rpt)

        @pl.when(cid == 0)
        def _():
            pltpu.sync_copy(acc.at[sl], out0.at[sl])

        @pl.when(cid == 1)
        def _():
            pltpu.sync_copy(acc.at[sl], out1.at[sl])

    return sck


_sc_segsum_36 = _make_sc_segsum(36)
_sc_segsum_8 = _make_sc_segsum(8)


def kernel(one_hot, features, gemme_features, a_res,
           enc_W0, enc_b0, enc_W1, enc_b1,
           conv0_Wc, conv0_Ws, conv0_b, conv1_Wc, conv1_Ws, conv1_b,
           sc_W0, sc_b0, sc_W1, sc_b1, sc_W2, sc_b2, fin_W, fin_b):
    f32 = jnp.float32
    w0a, w0b = enc_W0[:20], enc_W0[20:]
    wc0 = conv0_Wc.sum(axis=0)
    wc1 = conv1_Wc.sum(axis=0)
    row = lambda b: b.reshape(1, -1)

    oh = jnp.pad(one_hot, ((0, N_PAD - N), (0, 0)))
    ft = jnp.pad(features, ((0, N_PAD - N), (0, 0)))

    src = jnp.pad(a_res[0], (0, E_PAD - E)).reshape(E_PAD // CH, CH)
    dst = jnp.pad(a_res[1], (0, E_PAD - E), constant_values=N).reshape(
        E_PAD // CH, CH)

    m0, t0 = _tc_call(
        _enc_body,
        [oh, ft, w0a, w0b, row(enc_b0), enc_W1, row(enc_b1), wc0, conv0_Ws,
         row(conv0_b)],
        2, [36, 36])

    z36 = jnp.zeros((N_PAD, 36), f32)
    p0, p1 = _sc_segsum_36(m0, src, dst, z36)

    m1, t1 = _tc_call(
        _mid_body,
        [p0, p1, t0, wc1, conv1_Ws, row(conv1_b)],
        3, [8, 8])

    z8 = jnp.zeros((N_PAD, 8), f32)
    q0, q1 = _sc_segsum_8(m1, src, dst, z8)

    (out,) = _tc_call(
        _score_body,
        [q0, q1, t1, sc_W0, row(sc_b0), sc_W1, row(sc_b1), sc_W2, row(sc_b2),
         fin_W, row(fin_b)],
        3, [1])

    return out[:N]


# speedup vs baseline: 4.5650x; 4.5650x over previous
import functools
from functools import partial

import jax
import jax.numpy as jnp
from jax import lax
from jax.experimental import pallas as pl
from jax.experimental.pallas import tpu as pltpu
from jax.experimental.pallas import tpu_sc as plsc

N = 50000
E = 800000

RB = 512
N_PAD = 50176
G = N_PAD // RB

NC, NS = 2, 16
NW = NC * NS
CH = 128
E_PAD = 819200
NCH_W = E_PAD // NW // CH
NCH_S = E_PAD // NS // CH
DH0 = 32


def _dot(a, b):
    return jax.lax.dot(a, b, precision=jax.lax.Precision.HIGHEST,
                       preferred_element_type=jnp.float32)


def _elu(x):
    return jnp.where(x > 0, x, jnp.exp(x) - 1.0)


def _enc_body(oh, ft, w0a, w0b, b0, w1, b1, wc0a, wc0b, ws0, b0c,
              m0a, m0b, t0):
    h = _dot(oh[...], w0a[...]) + _dot(ft[...], w0b[...]) + b0[...]
    h = _elu(h)
    h = _elu(_dot(h, w1[...]) + b1[...])
    m0a[...] = _dot(h, wc0a[...])
    m0b[...] = _dot(h, wc0b[...])
    t0[...] = _dot(h, ws0[...]) + b0c[...]


def _mid_body(p0, p1, t0, wc1, ws1, b1c, m1, t1):
    x1 = _elu(jnp.concatenate([p0[:, :18], p1[:, :18]], axis=1) + t0[...])
    m1[...] = _dot(x1, wc1[...])
    t1[...] = _dot(x1, ws1[...]) + b1c[...]


def _score_body(q0, q1, t1, w0, b0, w1, b1, w2, b2, fw, fb, out):
    x2 = _elu(q0[...] + q1[...] + t1[...])
    h = _elu(_dot(x2, w0[...]) + b0[...])
    h = _elu(_dot(h, w1[...]) + b1[...])
    h = _elu(_dot(h, w2[...]) + b2[...])
    out[...] = jax.nn.sigmoid(_dot(h, fw[...]) + fb[...])


def _row_spec(d):
    return pl.BlockSpec((RB, d), lambda i: (i, 0))


def _full_spec(shape):
    return pl.BlockSpec(shape, lambda i: tuple(0 for _ in shape))


def _tc_call(body, in_arrays, n_row_args, out_widths):
    in_specs = [_row_spec(a.shape[1]) for a in in_arrays[:n_row_args]]
    in_specs += [_full_spec(a.shape) for a in in_arrays[n_row_args:]]
    out_shape = [jax.ShapeDtypeStruct((N_PAD, d), jnp.float32) for d in out_widths]
    out_specs = [_row_spec(d) for d in out_widths]
    return pl.pallas_call(
        body,
        grid=(G,),
        in_specs=in_specs,
        out_specs=out_specs,
        out_shape=out_shape,
        compiler_params=pltpu.CompilerParams(
            dimension_semantics=("arbitrary",)),
    )(*in_arrays)


_SC_PARAMS = pltpu.CompilerParams(use_tc_tiling_on_sc=False)


def _mesh():
    return plsc.VectorSubcoreMesh(core_axis_name="c", subcore_axis_name="s",
                                  num_cores=NC, num_subcores=NS)


BCH = 8
BE = BCH * CH


def _gather_scatter_loop(table, src2d, dst2d, cbase, srcv, dstv, rows, acc,
                         sem, nch):
    def block(r, carry):
        coff = cbase + r * BCH
        pltpu.sync_copy(src2d.at[pl.ds(coff, BCH)], srcv)
        pltpu.sync_copy(dst2d.at[pl.ds(coff, BCH)], dstv)

        def step(j, c):
            pltpu.async_copy(table.at[srcv.at[j]], rows, sem).wait()
            pltpu.sync_copy(rows, acc.at[dstv.at[j]], add=True)
            return c

        lax.fori_loop(0, BCH, step, 0, unroll=False)
        return carry

    lax.fori_loop(0, nch // BCH, block, 0, unroll=False)


def _make_sc_conv0(dh):
    rpt = N_PAD // NS
    eps = E_PAD // NS

    @functools.partial(
        pl.kernel,
        mesh=_mesh(),
        out_type=[jax.ShapeDtypeStruct((N_PAD, dh), jnp.float32)] * 2,
        scratch_types=[
            pltpu.VMEM((BCH, CH), jnp.int32),
            pltpu.VMEM((BCH, CH), jnp.int32),
            pltpu.VMEM((CH, dh), jnp.float32),
            pltpu.VMEM_SHARED((N_PAD, dh), jnp.float32),
            pltpu.SemaphoreType.DMA,
        ],
        compiler_params=_SC_PARAMS,
    )
    def sck(ta, tb, src1d, dst1d, zeros, out0, out1, srcv, dstv, rows, acc,
            sem):
        cid = lax.axis_index("c")
        sid = lax.axis_index("s")
        sl = pl.ds(sid * rpt, rpt)

        pltpu.sync_copy(zeros.at[sl], acc.at[sl])
        plsc.subcore_barrier()
        cbase = sid * NCH_S

        @pl.when(cid == 0)
        def _():
            _gather_scatter_loop(ta, src1d, dst1d, cbase, srcv, dstv, rows,
                                 acc, sem, NCH_S)

        @pl.when(cid == 1)
        def _():
            _gather_scatter_loop(tb, src1d, dst1d, cbase, srcv, dstv, rows,
                                 acc, sem, NCH_S)

        plsc.subcore_barrier()

        @pl.when(cid == 0)
        def _():
            pltpu.sync_copy(acc.at[sl], out0.at[sl])

        @pl.when(cid == 1)
        def _():
            pltpu.sync_copy(acc.at[sl], out1.at[sl])

    return sck


def _make_sc_conv1(d):
    rpt = N_PAD // NS
    epw = E_PAD // NW

    @functools.partial(
        pl.kernel,
        mesh=_mesh(),
        out_type=[jax.ShapeDtypeStruct((N_PAD, d), jnp.float32)] * 2,
        scratch_types=[
            pltpu.VMEM((BCH, CH), jnp.int32),
            pltpu.VMEM((BCH, CH), jnp.int32),
            pltpu.VMEM((CH, d), jnp.float32),
            pltpu.VMEM_SHARED((N_PAD, d), jnp.float32),
            pltpu.SemaphoreType.DMA,
        ],
        compiler_params=_SC_PARAMS,
    )
    def sck(table, src1d, dst1d, zeros, out0, out1, srcv, dstv, rows, acc,
            sem):
        cid = lax.axis_index("c")
        sid = lax.axis_index("s")
        wid = cid * NS + sid
        sl = pl.ds(sid * rpt, rpt)

        pltpu.sync_copy(zeros.at[sl], acc.at[sl])
        plsc.subcore_barrier()

        _gather_scatter_loop(table, src1d, dst1d, wid * NCH_W, srcv, dstv,
                             rows, acc, sem, NCH_W)

        plsc.subcore_barrier()

        @pl.when(cid == 0)
        def _():
            pltpu.sync_copy(acc.at[sl], out0.at[sl])

        @pl.when(cid == 1)
        def _():
            pltpu.sync_copy(acc.at[sl], out1.at[sl])

    return sck


_sc_cache = {}


def _sc_conv0():
    if "c0" not in _sc_cache:
        _sc_cache["c0"] = _make_sc_conv0(DH0)
    return _sc_cache["c0"]


def _sc_conv1():
    if "c1" not in _sc_cache:
        _sc_cache["c1"] = _make_sc_conv1(8)
    return _sc_cache["c1"]


def kernel(one_hot, features, gemme_features, a_res,
           enc_W0, enc_b0, enc_W1, enc_b1,
           conv0_Wc, conv0_Ws, conv0_b, conv1_Wc, conv1_Ws, conv1_b,
           sc_W0, sc_b0, sc_W1, sc_b1, sc_W2, sc_b2, fin_W, fin_b):
    f32 = jnp.float32
    w0a, w0b = enc_W0[:20], enc_W0[20:]
    wc0 = conv0_Wc.sum(axis=0)
    pad_w = ((0, 0), (0, DH0 - 18))
    wc0a = jnp.pad(wc0[:, :18], pad_w)
    wc0b = jnp.pad(wc0[:, 18:], pad_w)
    wc1 = conv1_Wc.sum(axis=0)
    row = lambda b: b.reshape(1, -1)

    oh = jnp.pad(one_hot, ((0, N_PAD - N), (0, 0)))
    ft = jnp.pad(features, ((0, N_PAD - N), (0, 0)))

    src = jnp.pad(a_res[0], (0, E_PAD - E)).reshape(E_PAD // CH, CH)
    dst = jnp.pad(a_res[1], (0, E_PAD - E), constant_values=N).reshape(
        E_PAD // CH, CH)

    m0a, m0b, t0 = _tc_call(
        _enc_body,
        [oh, ft, w0a, w0b, row(enc_b0), enc_W1, row(enc_b1), wc0a, wc0b,
         conv0_Ws, row(conv0_b)],
        2, [DH0, DH0, 36])

    z18 = jnp.zeros((N_PAD, DH0), f32)
    p0, p1 = _sc_conv0()(m0a, m0b, src, dst, z18)

    m1, t1 = _tc_call(
        _mid_body,
        [p0, p1, t0, wc1, conv1_Ws, row(conv1_b)],
        3, [8, 8])

    z8 = jnp.zeros((N_PAD, 8), f32)
    q0, q1 = _sc_conv1()(m1, src, dst, z8)

    (out,) = _tc_call(
        _score_body,
        [q0, q1, t1, sc_W0, row(sc_b0), sc_W1, row(sc_b1), sc_W2, row(sc_b2),
         fin_W, row(fin_b)],
        3, [1])

    return out[:N]


# speedup vs baseline: 5.7277x; 1.2547x over previous
import functools
from functools import partial

import jax
import jax.numpy as jnp
from jax import lax
from jax.experimental import pallas as pl
from jax.experimental.pallas import tpu as pltpu
from jax.experimental.pallas import tpu_sc as plsc

N = 50000
E = 800000

RB = 512
N_PAD = 50176
G = N_PAD // RB

NC, NS = 2, 16
NW = NC * NS
CH = 128
E_PAD = 819200
NCH_W = E_PAD // NW // CH
NCH_S = E_PAD // NS // CH
DH0 = 32


def _dot(a, b):
    return jax.lax.dot(a, b, precision=jax.lax.Precision.HIGHEST,
                       preferred_element_type=jnp.float32)


def _elu(x):
    return jnp.where(x > 0, x, jnp.exp(x) - 1.0)


def _enc_body(oh, ft, w0a, w0b, b0, w1, b1, wc0a, wc0b, ws0, b0c,
              m0a, m0b, t0):
    h = _dot(oh[...], w0a[...]) + _dot(ft[...], w0b[...]) + b0[...]
    h = _elu(h)
    h = _elu(_dot(h, w1[...]) + b1[...])
    m0a[...] = _dot(h, wc0a[...])
    m0b[...] = _dot(h, wc0b[...])
    t0[...] = _dot(h, ws0[...]) + b0c[...]


def _mid_body(p0, p1, t0, wc1, ws1, b1c, m1, t1):
    x1 = _elu(jnp.concatenate([p0[:, :18], p1[:, :18]], axis=1) + t0[...])
    m1[...] = _dot(x1, wc1[...])
    t1[...] = _dot(x1, ws1[...]) + b1c[...]


def _score_body(q0, q1, t1, w0, b0, w1, b1, w2, b2, fw, fb, out):
    x2 = _elu(q0[...] + q1[...] + t1[...])
    h = _elu(_dot(x2, w0[...]) + b0[...])
    h = _elu(_dot(h, w1[...]) + b1[...])
    h = _elu(_dot(h, w2[...]) + b2[...])
    out[...] = jax.nn.sigmoid(_dot(h, fw[...]) + fb[...])


def _row_spec(d):
    return pl.BlockSpec((RB, d), lambda i: (i, 0))


def _full_spec(shape):
    return pl.BlockSpec(shape, lambda i: tuple(0 for _ in shape))


def _tc_call(body, in_arrays, n_row_args, out_widths):
    in_specs = [_row_spec(a.shape[1]) for a in in_arrays[:n_row_args]]
    in_specs += [_full_spec(a.shape) for a in in_arrays[n_row_args:]]
    out_shape = [jax.ShapeDtypeStruct((N_PAD, d), jnp.float32) for d in out_widths]
    out_specs = [_row_spec(d) for d in out_widths]
    return pl.pallas_call(
        body,
        grid=(G,),
        in_specs=in_specs,
        out_specs=out_specs,
        out_shape=out_shape,
        compiler_params=pltpu.CompilerParams(
            dimension_semantics=("arbitrary",)),
    )(*in_arrays)


_SC_PARAMS = pltpu.CompilerParams(use_tc_tiling_on_sc=False)


def _mesh():
    return plsc.VectorSubcoreMesh(core_axis_name="c", subcore_axis_name="s",
                                  num_cores=NC, num_subcores=NS)


BCH0, NB0 = 20, 5
BCH1, NB1 = 8, 8


def _gather_scatter_loop(table, src2d, dst2d, cbase, srcv, dstv, rows, acc,
                         gsems, ssems, nch, bch, nb):
    def block(r, carry):
        coff = cbase + r * bch
        pltpu.sync_copy(src2d.at[pl.ds(coff, bch)], srcv)
        pltpu.sync_copy(dst2d.at[pl.ds(coff, bch)], dstv)

        desc_g = [None] * nb
        desc_s = [None] * nb

        def scatter(jj):
            b = jj % nb
            desc_g[b].wait()
            desc_s[b] = pltpu.async_copy(
                rows.at[b], acc.at[dstv.at[jj]], ssems.at[b], add=True)

        for j in range(bch):
            b = j % nb
            if desc_s[b] is not None:
                desc_s[b].wait()
            desc_g[b] = pltpu.async_copy(
                table.at[srcv.at[j]], rows.at[b], gsems.at[b])
            if j - nb + 1 >= 0:
                scatter(j - nb + 1)
        for jj in range(max(bch - nb + 1, 0), bch):
            scatter(jj)
        for b in range(nb):
            if desc_s[b] is not None:
                desc_s[b].wait()
        return carry

    lax.fori_loop(0, nch // bch, block, 0, unroll=False)


def _make_sc_conv0(dh):
    rpt = N_PAD // NS
    eps = E_PAD // NS

    @functools.partial(
        pl.kernel,
        mesh=_mesh(),
        out_type=[jax.ShapeDtypeStruct((N_PAD, dh), jnp.float32)] * 2,
        scratch_types=[
            pltpu.VMEM((BCH0, CH), jnp.int32),
            pltpu.VMEM((BCH0, CH), jnp.int32),
            pltpu.VMEM((NB0, CH, dh), jnp.float32),
            pltpu.VMEM_SHARED((N_PAD, dh), jnp.float32),
            pltpu.SemaphoreType.DMA((NB0,)),
            pltpu.SemaphoreType.DMA((NB0,)),
        ],
        compiler_params=_SC_PARAMS,
    )
    def sck(ta, tb, src1d, dst1d, zeros, out0, out1, srcv, dstv, rows, acc,
            gsems, ssems):
        cid = lax.axis_index("c")
        sid = lax.axis_index("s")
        sl = pl.ds(sid * rpt, rpt)

        pltpu.sync_copy(zeros.at[sl], acc.at[sl])
        plsc.subcore_barrier()
        cbase = sid * NCH_S

        @pl.when(cid == 0)
        def _():
            _gather_scatter_loop(ta, src1d, dst1d, cbase, srcv, dstv, rows,
                                 acc, gsems, ssems, NCH_S, BCH0, NB0)

        @pl.when(cid == 1)
        def _():
            _gather_scatter_loop(tb, src1d, dst1d, cbase, srcv, dstv, rows,
                                 acc, gsems, ssems, NCH_S, BCH0, NB0)

        plsc.subcore_barrier()

        @pl.when(cid == 0)
        def _():
            pltpu.sync_copy(acc.at[sl], out0.at[sl])

        @pl.when(cid == 1)
        def _():
            pltpu.sync_copy(acc.at[sl], out1.at[sl])

    return sck


def _make_sc_conv1(d):
    rpt = N_PAD // NS
    epw = E_PAD // NW

    @functools.partial(
        pl.kernel,
        mesh=_mesh(),
        out_type=[jax.ShapeDtypeStruct((N_PAD, d), jnp.float32)] * 2,
        scratch_types=[
            pltpu.VMEM((BCH1, CH), jnp.int32),
            pltpu.VMEM((BCH1, CH), jnp.int32),
            pltpu.VMEM((NB1, CH, d), jnp.float32),
            pltpu.VMEM_SHARED((N_PAD, d), jnp.float32),
            pltpu.SemaphoreType.DMA((NB1,)),
            pltpu.SemaphoreType.DMA((NB1,)),
        ],
        compiler_params=_SC_PARAMS,
    )
    def sck(table, src1d, dst1d, zeros, out0, out1, srcv, dstv, rows, acc,
            gsems, ssems):
        cid = lax.axis_index("c")
        sid = lax.axis_index("s")
        wid = cid * NS + sid
        sl = pl.ds(sid * rpt, rpt)

        pltpu.sync_copy(zeros.at[sl], acc.at[sl])
        plsc.subcore_barrier()

        _gather_scatter_loop(table, src1d, dst1d, wid * NCH_W, srcv, dstv,
                             rows, acc, gsems, ssems, NCH_W, BCH1, NB1)

        plsc.subcore_barrier()

        @pl.when(cid == 0)
        def _():
            pltpu.sync_copy(acc.at[sl], out0.at[sl])

        @pl.when(cid == 1)
        def _():
            pltpu.sync_copy(acc.at[sl], out1.at[sl])

    return sck


_sc_cache = {}


def _sc_conv0():
    if "c0" not in _sc_cache:
        _sc_cache["c0"] = _make_sc_conv0(DH0)
    return _sc_cache["c0"]


def _sc_conv1():
    if "c1" not in _sc_cache:
        _sc_cache["c1"] = _make_sc_conv1(8)
    return _sc_cache["c1"]


def kernel(one_hot, features, gemme_features, a_res,
           enc_W0, enc_b0, enc_W1, enc_b1,
           conv0_Wc, conv0_Ws, conv0_b, conv1_Wc, conv1_Ws, conv1_b,
           sc_W0, sc_b0, sc_W1, sc_b1, sc_W2, sc_b2, fin_W, fin_b):
    f32 = jnp.float32
    w0a, w0b = enc_W0[:20], enc_W0[20:]
    wc0 = conv0_Wc.sum(axis=0)
    pad_w = ((0, 0), (0, DH0 - 18))
    wc0a = jnp.pad(wc0[:, :18], pad_w)
    wc0b = jnp.pad(wc0[:, 18:], pad_w)
    wc1 = conv1_Wc.sum(axis=0)
    row = lambda b: b.reshape(1, -1)

    oh = jnp.pad(one_hot, ((0, N_PAD - N), (0, 0)))
    ft = jnp.pad(features, ((0, N_PAD - N), (0, 0)))

    src = jnp.pad(a_res[0], (0, E_PAD - E)).reshape(E_PAD // CH, CH)
    dst = jnp.pad(a_res[1], (0, E_PAD - E), constant_values=N).reshape(
        E_PAD // CH, CH)

    m0a, m0b, t0 = _tc_call(
        _enc_body,
        [oh, ft, w0a, w0b, row(enc_b0), enc_W1, row(enc_b1), wc0a, wc0b,
         conv0_Ws, row(conv0_b)],
        2, [DH0, DH0, 36])

    z18 = jnp.zeros((N_PAD, DH0), f32)
    p0, p1 = _sc_conv0()(m0a, m0b, src, dst, z18)

    m1, t1 = _tc_call(
        _mid_body,
        [p0, p1, t0, wc1, conv1_Ws, row(conv1_b)],
        3, [8, 8])

    z8 = jnp.zeros((N_PAD, 8), f32)
    q0, q1 = _sc_conv1()(m1, src, dst, z8)

    (out,) = _tc_call(
        _score_body,
        [q0, q1, t1, sc_W0, row(sc_b0), sc_W1, row(sc_b1), sc_W2, row(sc_b2),
         fin_W, row(fin_b)],
        3, [1])

    return out[:N]


# speedup vs baseline: 6.3301x; 1.1052x over previous
import functools

import jax
import jax.numpy as jnp
from jax import lax
from jax.experimental import pallas as pl
from jax.experimental.pallas import tpu as pltpu
from jax.experimental.pallas import tpu_sc as plsc

N = 50000
E = 800000

RB = 512
N_PAD = 50176
G = N_PAD // RB

NC, NS = 2, 16
NW = NC * NS
CH = 128
E_PAD = 819200
NCH_W = E_PAD // NW // CH
NCH_S = E_PAD // NS // CH
DH0 = 32

BCH0, NB0 = 20, 5
BCH1, NB1 = 8, 8


def _dot(a, b):
    return jax.lax.dot(a, b, precision=jax.lax.Precision.HIGHEST,
                       preferred_element_type=jnp.float32)


def _elu(x):
    return jnp.where(x > 0, x, jnp.exp(x) - 1.0)


def _enc_body(oh, ft, w0a, w0b, b0, w1, b1, wc0a, wc0b, ws0, b0c, eo):
    h = _elu(_dot(oh[...], w0a[...]) + _dot(ft[...], w0b[...]) + b0[...])
    h = _elu(_dot(h, w1[...]) + b1[...])
    m0a = _dot(h, wc0a[...])
    m0b = _dot(h, wc0b[...])
    t0 = _dot(h, ws0[...]) + b0c[...]
    eo[...] = jnp.concatenate(
        [m0a, m0b, t0, jnp.zeros((RB, 28), jnp.float32)], axis=1)


def _mid_body(eo, p, wc1, ws1, b1c, mo):
    agg = jnp.concatenate([p[:, 0:18], p[:, 32:50]], axis=1)
    x1 = _elu(agg + eo[:, 64:100])
    m1 = _dot(x1, wc1[...])
    t1 = _dot(x1, ws1[...]) + b1c[...]
    mo[...] = jnp.concatenate(
        [m1, t1, jnp.zeros((RB, 112), jnp.float32)], axis=1)


def _score_body(mo, q, w0, b0, w1, b1, w2, b2, fw, fb, out):
    x2 = _elu(q[:, 0:8] + q[:, 16:24] + mo[:, 8:16])
    h = _elu(_dot(x2, w0[...]) + b0[...])
    h = _elu(_dot(h, w1[...]) + b1[...])
    h = _elu(_dot(h, w2[...]) + b2[...])
    out[...] = jax.nn.sigmoid(_dot(h, fw[...]) + fb[...])


def _row_spec(d):
    return pl.BlockSpec((RB, d), lambda i: (i, 0))


def _full_spec(shape):
    return pl.BlockSpec(shape, lambda i: tuple(0 for _ in shape))


def _tc_call(body, in_arrays, n_row_args, out_widths):
    in_specs = [_row_spec(a.shape[1]) for a in in_arrays[:n_row_args]]
    in_specs += [_full_spec(a.shape) for a in in_arrays[n_row_args:]]
    out_shape = [jax.ShapeDtypeStruct((N_PAD, d), jnp.float32)
                 for d in out_widths]
    out_specs = [_row_spec(d) for d in out_widths]
    return pl.pallas_call(
        body,
        grid=(G,),
        in_specs=in_specs,
        out_specs=out_specs,
        out_shape=out_shape,
        compiler_params=pltpu.CompilerParams(
            dimension_semantics=("arbitrary",)),
    )(*in_arrays)


_SC_PARAMS = pltpu.CompilerParams(use_tc_tiling_on_sc=False)
RPT = N_PAD // NS
ZR = 128


def _mesh():
    return plsc.VectorSubcoreMesh(core_axis_name="c", subcore_axis_name="s",
                                  num_cores=NC, num_subcores=NS)


def _row_blocks(nrows):
    nfull, rem = divmod(nrows, ZR)
    return [(k * ZR, ZR) for k in range(nfull)] + ([(nfull * ZR, rem)]
                                                   if rem else [])


def _extract_loop(packed, co, w, t_out, base, nrows, rows, gsems, ssems, nb):
    blocks = _row_blocks(nrows)
    desc_r = [None] * nb
    desc_w = [None] * nb

    def write(kk):
        b = kk % nb
        desc_r[b].wait()
        off, sz = blocks[kk]
        desc_w[b] = pltpu.async_copy(
            rows.at[b, pl.ds(0, sz)],
            t_out.at[pl.ds(base + off, sz)], ssems.at[b])

    for k, (off, sz) in enumerate(blocks):
        b = k % nb
        if desc_w[b] is not None:
            desc_w[b].wait()
        desc_r[b] = pltpu.async_copy(
            packed.at[pl.ds(base + off, sz), pl.ds(co, w)],
            rows.at[b, pl.ds(0, sz)], gsems.at[b])
        if k - nb + 1 >= 0:
            write(k - nb + 1)
    for kk in range(max(len(blocks) - nb + 1, 0), len(blocks)):
        write(kk)
    for b in range(nb):
        if desc_w[b] is not None:
            desc_w[b].wait()


def _zero_acc(rows, acc, sid, d, sem):
    zv = jnp.zeros((16,), jnp.float32)

    def zrow(i, c):
        for k in range(d // 16):
            rows[0, i, pl.ds(k * 16, 16)] = zv
        return c

    lax.fori_loop(0, ZR, zrow, 0, unroll=False)
    base = sid * RPT
    descs = []
    for off, sz in _row_blocks(RPT):
        descs.append(pltpu.async_copy(
            rows.at[0, pl.ds(0, sz)], acc.at[pl.ds(base + off, sz)], sem))
    for dsc in descs:
        dsc.wait()


def _gather_scatter_loop(table, src2d, dst2d, cbase, srcv, dstv, rows,
                         acc, gsems, ssems, nch, bch, nb):
    def block(r, carry):
        coff = cbase + r * bch
        pltpu.sync_copy(src2d.at[pl.ds(coff, bch)], srcv)
        pltpu.sync_copy(dst2d.at[pl.ds(coff, bch)], dstv)

        desc_g = [None] * nb
        desc_s = [None] * nb

        def scatter(jj):
            b = jj % nb
            desc_g[b].wait()
            desc_s[b] = pltpu.async_copy(
                rows.at[b], acc.at[dstv.at[jj]], ssems.at[b], add=True)

        for j in range(bch):
            b = j % nb
            if desc_s[b] is not None:
                desc_s[b].wait()
            desc_g[b] = pltpu.async_copy(
                table.at[srcv.at[j]], rows.at[b], gsems.at[b])
            if j - nb + 1 >= 0:
                scatter(j - nb + 1)
        for jj in range(max(bch - nb + 1, 0), bch):
            scatter(jj)
        for b in range(nb):
            if desc_s[b] is not None:
                desc_s[b].wait()
        return carry

    lax.fori_loop(0, nch // bch, block, 0, unroll=False)


def _make_sc_conv0():
    @functools.partial(
        pl.kernel,
        mesh=_mesh(),
        out_type=[jax.ShapeDtypeStruct((N_PAD, 128), jnp.float32),
                  jax.ShapeDtypeStruct((N_PAD, DH0), jnp.float32),
                  jax.ShapeDtypeStruct((N_PAD, DH0), jnp.float32)],
        scratch_types=[
            pltpu.VMEM((BCH0, CH), jnp.int32),
            pltpu.VMEM((BCH0, CH), jnp.int32),
            pltpu.VMEM((NB0, CH, DH0), jnp.float32),
            pltpu.VMEM_SHARED((N_PAD, DH0), jnp.float32),
            pltpu.SemaphoreType.DMA((NB0,)),
            pltpu.SemaphoreType.DMA((NB0,)),
        ],
        compiler_params=_SC_PARAMS,
    )
    def sck(table, src2d, dst2d, out, ta, tb, srcv, dstv, rows, acc,
            gsems, ssems):
        cid = lax.axis_index("c")
        sid = lax.axis_index("s")
        sl = pl.ds(sid * RPT, RPT)
        cbase = sid * NCH_S

        def run(co, t_out):
            _extract_loop(table, co, DH0, t_out, sid * RPT, RPT, rows,
                          gsems, ssems, NB0)
            _zero_acc(rows, acc, sid, DH0, gsems.at[0])
            plsc.subcore_barrier()
            _gather_scatter_loop(t_out, src2d, dst2d, cbase, srcv, dstv,
                                 rows, acc, gsems, ssems, NCH_S, BCH0, NB0)
            plsc.subcore_barrier()
            pltpu.sync_copy(acc.at[sl], out.at[sl, pl.ds(co, DH0)])

        @pl.when(cid == 0)
        def _():
            run(0, ta)

        @pl.when(cid == 1)
        def _():
            run(32, tb)

    return sck


def _make_sc_conv1():
    d = 16

    @functools.partial(
        pl.kernel,
        mesh=_mesh(),
        out_type=[jax.ShapeDtypeStruct((N_PAD, 128), jnp.float32),
                  jax.ShapeDtypeStruct((N_PAD, d), jnp.float32),
                  jax.ShapeDtypeStruct((N_PAD, d), jnp.float32)],
        scratch_types=[
            pltpu.VMEM((BCH1, CH), jnp.int32),
            pltpu.VMEM((BCH1, CH), jnp.int32),
            pltpu.VMEM((NB1, CH, d), jnp.float32),
            pltpu.VMEM_SHARED((N_PAD, d), jnp.float32),
            pltpu.SemaphoreType.DMA((NB1,)),
            pltpu.SemaphoreType.DMA((NB1,)),
        ],
        compiler_params=_SC_PARAMS,
    )
    def sck(table, src2d, dst2d, out, ta, tb, srcv, dstv, rows, acc,
            gsems, ssems):
        cid = lax.axis_index("c")
        sid = lax.axis_index("s")
        wid = cid * NS + sid
        sl = pl.ds(sid * RPT, RPT)

        def run(co, t_out):
            _extract_loop(table, 0, d, t_out, sid * RPT, RPT, rows,
                          gsems, ssems, NB1)
            _zero_acc(rows, acc, sid, d, gsems.at[0])
            plsc.subcore_barrier()
            _gather_scatter_loop(t_out, src2d, dst2d, wid * NCH_W, srcv,
                                 dstv, rows, acc, gsems, ssems,
                                 NCH_W, BCH1, NB1)
            plsc.subcore_barrier()
            pltpu.sync_copy(acc.at[sl], out.at[sl, pl.ds(co, d)])

        @pl.when(cid == 0)
        def _():
            run(0, ta)

        @pl.when(cid == 1)
        def _():
            run(16, tb)

    return sck


_sc_cache = {}


def _sc_conv0():
    if "c0" not in _sc_cache:
        _sc_cache["c0"] = _make_sc_conv0()
    return _sc_cache["c0"]


def _sc_conv1():
    if "c1" not in _sc_cache:
        _sc_cache["c1"] = _make_sc_conv1()
    return _sc_cache["c1"]


def kernel(one_hot, features, gemme_features, a_res,
           enc_W0, enc_b0, enc_W1, enc_b1,
           conv0_Wc, conv0_Ws, conv0_b, conv1_Wc, conv1_Ws, conv1_b,
           sc_W0, sc_b0, sc_W1, sc_b1, sc_W2, sc_b2, fin_W, fin_b):
    w0a, w0b = enc_W0[:20], enc_W0[20:]
    wc0 = conv0_Wc.sum(axis=0)
    pad_w = ((0, 0), (0, DH0 - 18))
    wc0a = jnp.pad(wc0[:, :18], pad_w)
    wc0b = jnp.pad(wc0[:, 18:], pad_w)
    wc1 = conv1_Wc.sum(axis=0)
    row = lambda b: b.reshape(1, -1)

    src = jnp.pad(a_res[0], (0, E_PAD - E)).reshape(E_PAD // CH, CH)
    dst = jnp.pad(a_res[1], (0, E_PAD - E), constant_values=N).reshape(
        E_PAD // CH, CH)

    (eo,) = _tc_call(
        _enc_body,
        [one_hot, features, w0a, w0b, row(enc_b0), enc_W1, row(enc_b1),
         wc0a, wc0b, conv0_Ws, row(conv0_b)],
        2, [128])

    p, _, _ = _sc_conv0()(eo, src, dst)

    (mo,) = _tc_call(
        _mid_body,
        [eo, p, wc1, conv1_Ws, row(conv1_b)],
        2, [128])

    q, _, _ = _sc_conv1()(mo, src, dst)

    (out,) = _tc_call(
        _score_body,
        [mo, q, sc_W0, row(sc_b0), sc_W1, row(sc_b1), sc_W2, row(sc_b2),
         fin_W, row(fin_b)],
        2, [1])

    return out[:N]


# speedup vs baseline: 9.2016x; 1.4536x over previous
import functools

import jax
import jax.numpy as jnp
from jax import lax
from jax.experimental import pallas as pl
from jax.experimental.pallas import tpu as pltpu
from jax.experimental.pallas import tpu_sc as plsc

N = 50000
E = 800000

RB = 1024
N_PAD = 50176
G = N_PAD // RB

NC, NS = 2, 16
NW = NC * NS
CH = 128
E_PAD = 819200
NCH_W = E_PAD // NW // CH
NCH_S = E_PAD // NS // CH
DH0 = 32

BCH0, NB0 = 20, 5
BCH1, NB1 = 8, 8


def _dot(a, b):
    return jax.lax.dot(a, b, preferred_element_type=jnp.float32)


def _elu(x):
    return jnp.where(x > 0, x, jnp.exp(x) - 1.0)


def _enc_body(oh, ft, w0a, w0b, b0, w1, b1, wc0a, wc0b, ws0, b0c, eo):
    h = _elu(_dot(oh[...], w0a[...]) + _dot(ft[...], w0b[...]) + b0[...])
    h = _elu(_dot(h, w1[...]) + b1[...])
    m0a = _dot(h, wc0a[...])
    m0b = _dot(h, wc0b[...])
    t0 = _dot(h, ws0[...]) + b0c[...]
    eo[...] = jnp.concatenate(
        [m0a, m0b, t0, jnp.zeros((RB, 28), jnp.float32)], axis=1)


def _mid_body(eo, p, wc1, ws1, b1c, mo):
    agg = jnp.concatenate([p[:, 0:18], p[:, 32:50]], axis=1)
    x1 = _elu(agg + eo[:, 64:100])
    m1 = _dot(x1, wc1[...])
    t1 = _dot(x1, ws1[...]) + b1c[...]
    mo[...] = jnp.concatenate(
        [m1, t1, jnp.zeros((RB, 112), jnp.float32)], axis=1)


def _score_body(mo, q, w0, b0, w1, b1, w2, b2, fw, fb, out):
    x2 = _elu(q[:, 0:8] + q[:, 16:24] + mo[:, 8:16])
    h = _elu(_dot(x2, w0[...]) + b0[...])
    h = _elu(_dot(h, w1[...]) + b1[...])
    h = _elu(_dot(h, w2[...]) + b2[...])
    out[...] = jax.nn.sigmoid(_dot(h, fw[...]) + fb[...])


def _row_spec(d):
    return pl.BlockSpec((RB, d), lambda i: (i, 0))


def _full_spec(shape):
    return pl.BlockSpec(shape, lambda i: tuple(0 for _ in shape))


def _tc_call(body, in_arrays, n_row_args, out_widths):
    in_specs = [_row_spec(a.shape[1]) for a in in_arrays[:n_row_args]]
    in_specs += [_full_spec(a.shape) for a in in_arrays[n_row_args:]]
    out_shape = [jax.ShapeDtypeStruct((N_PAD, d), jnp.float32)
                 for d in out_widths]
    out_specs = [_row_spec(d) for d in out_widths]
    return pl.pallas_call(
        body,
        grid=(G,),
        in_specs=in_specs,
        out_specs=out_specs,
        out_shape=out_shape,
        compiler_params=pltpu.CompilerParams(
            dimension_semantics=("parallel",)),
    )(*in_arrays)


_SC_PARAMS = pltpu.CompilerParams(use_tc_tiling_on_sc=False)
RPT = N_PAD // NS
ZR = 128


def _mesh():
    return plsc.VectorSubcoreMesh(core_axis_name="c", subcore_axis_name="s",
                                  num_cores=NC, num_subcores=NS)


def _row_blocks(nrows):
    nfull, rem = divmod(nrows, ZR)
    return [(k * ZR, ZR) for k in range(nfull)] + ([(nfull * ZR, rem)]
                                                   if rem else [])


def _extract_loop(packed, co, w, t_out, base, nrows, rows, gsems, ssems, nb):
    blocks = _row_blocks(nrows)
    desc_r = [None] * nb
    desc_w = [None] * nb

    def write(kk):
        b = kk % nb
        desc_r[b].wait()
        off, sz = blocks[kk]
        desc_w[b] = pltpu.async_copy(
            rows.at[b, pl.ds(0, sz)],
            t_out.at[pl.ds(base + off, sz)], ssems.at[b])

    for k, (off, sz) in enumerate(blocks):
        b = k % nb
        if desc_w[b] is not None:
            desc_w[b].wait()
        desc_r[b] = pltpu.async_copy(
            packed.at[pl.ds(base + off, sz), pl.ds(co, w)],
            rows.at[b, pl.ds(0, sz)], gsems.at[b])
        if k - nb + 1 >= 0:
            write(k - nb + 1)
    for kk in range(max(len(blocks) - nb + 1, 0), len(blocks)):
        write(kk)
    for b in range(nb):
        if desc_w[b] is not None:
            desc_w[b].wait()


def _zero_acc(rows, acc, sid, d, sem):
    zv = jnp.zeros((16,), jnp.float32)

    def zrow(i, c):
        for k in range(d // 16):
            rows[0, i, pl.ds(k * 16, 16)] = zv
        return c

    lax.fori_loop(0, ZR, zrow, 0, unroll=False)
    base = sid * RPT
    descs = []
    for off, sz in _row_blocks(RPT):
        descs.append(pltpu.async_copy(
            rows.at[0, pl.ds(0, sz)], acc.at[pl.ds(base + off, sz)], sem))
    for dsc in descs:
        dsc.wait()


def _gather_scatter_loop(table, src2d, dst2d, cbase, srcv, dstv, rows,
                         acc, gsems, ssems, nch, bch, nb):
    def block(r, carry):
        coff = cbase + r * bch
        pltpu.sync_copy(src2d.at[pl.ds(coff, bch)], srcv)
        pltpu.sync_copy(dst2d.at[pl.ds(coff, bch)], dstv)

        desc_g = [None] * nb
        desc_s = [None] * nb

        def scatter(jj):
            b = jj % nb
            desc_g[b].wait()
            desc_s[b] = pltpu.async_copy(
                rows.at[b], acc.at[dstv.at[jj]], ssems.at[b], add=True)

        for j in range(bch):
            b = j % nb
            if desc_s[b] is not None:
                desc_s[b].wait()
            desc_g[b] = pltpu.async_copy(
                table.at[srcv.at[j]], rows.at[b], gsems.at[b])
            if j - nb + 1 >= 0:
                scatter(j - nb + 1)
        for jj in range(max(bch - nb + 1, 0), bch):
            scatter(jj)
        for b in range(nb):
            if desc_s[b] is not None:
                desc_s[b].wait()
        return carry

    lax.fori_loop(0, nch // bch, block, 0, unroll=False)


def _make_sc_conv0():
    @functools.partial(
        pl.kernel,
        mesh=_mesh(),
        out_type=[jax.ShapeDtypeStruct((N_PAD, 128), jnp.float32),
                  jax.ShapeDtypeStruct((N_PAD, DH0), jnp.float32),
                  jax.ShapeDtypeStruct((N_PAD, DH0), jnp.float32)],
        scratch_types=[
            pltpu.VMEM((BCH0, CH), jnp.int32),
            pltpu.VMEM((BCH0, CH), jnp.int32),
            pltpu.VMEM((NB0, CH, DH0), jnp.float32),
            pltpu.VMEM_SHARED((N_PAD, DH0), jnp.float32),
            pltpu.SemaphoreType.DMA((NB0,)),
            pltpu.SemaphoreType.DMA((NB0,)),
        ],
        compiler_params=_SC_PARAMS,
    )
    def sck(table, src2d, dst2d, out, ta, tb, srcv, dstv, rows, acc,
            gsems, ssems):
        cid = lax.axis_index("c")
        sid = lax.axis_index("s")
        sl = pl.ds(sid * RPT, RPT)
        cbase = sid * NCH_S

        def run(co, t_out):
            _extract_loop(table, co, DH0, t_out, sid * RPT, RPT, rows,
                          gsems, ssems, NB0)
            _zero_acc(rows, acc, sid, DH0, gsems.at[0])
            plsc.subcore_barrier()
            _gather_scatter_loop(t_out, src2d, dst2d, cbase, srcv, dstv,
                                 rows, acc, gsems, ssems, NCH_S, BCH0, NB0)
            plsc.subcore_barrier()
            pltpu.sync_copy(acc.at[sl], out.at[sl, pl.ds(co, DH0)])

        @pl.when(cid == 0)
        def _():
            run(0, ta)

        @pl.when(cid == 1)
        def _():
            run(32, tb)

    return sck


def _make_sc_conv1():
    d = 16

    @functools.partial(
        pl.kernel,
        mesh=_mesh(),
        out_type=[jax.ShapeDtypeStruct((N_PAD, 128), jnp.float32),
                  jax.ShapeDtypeStruct((N_PAD, d), jnp.float32),
                  jax.ShapeDtypeStruct((N_PAD, d), jnp.float32)],
        scratch_types=[
            pltpu.VMEM((BCH1, CH), jnp.int32),
            pltpu.VMEM((BCH1, CH), jnp.int32),
            pltpu.VMEM((NB1, CH, d), jnp.float32),
            pltpu.VMEM_SHARED((N_PAD, d), jnp.float32),
            pltpu.SemaphoreType.DMA((NB1,)),
            pltpu.SemaphoreType.DMA((NB1,)),
        ],
        compiler_params=_SC_PARAMS,
    )
    def sck(table, src2d, dst2d, out, ta, tb, srcv, dstv, rows, acc,
            gsems, ssems):
        cid = lax.axis_index("c")
        sid = lax.axis_index("s")
        wid = cid * NS + sid
        sl = pl.ds(sid * RPT, RPT)

        def run(co, t_out):
            _extract_loop(table, 0, d, t_out, sid * RPT, RPT, rows,
                          gsems, ssems, NB1)
            _zero_acc(rows, acc, sid, d, gsems.at[0])
            plsc.subcore_barrier()
            _gather_scatter_loop(t_out, src2d, dst2d, wid * NCH_W, srcv,
                                 dstv, rows, acc, gsems, ssems,
                                 NCH_W, BCH1, NB1)
            plsc.subcore_barrier()
            pltpu.sync_copy(acc.at[sl], out.at[sl, pl.ds(co, d)])

        @pl.when(cid == 0)
        def _():
            run(0, ta)

        @pl.when(cid == 1)
        def _():
            run(16, tb)

    return sck


_sc_cache = {}


def _sc_conv0():
    if "c0" not in _sc_cache:
        _sc_cache["c0"] = _make_sc_conv0()
    return _sc_cache["c0"]


def _sc_conv1():
    if "c1" not in _sc_cache:
        _sc_cache["c1"] = _make_sc_conv1()
    return _sc_cache["c1"]


def kernel(one_hot, features, gemme_features, a_res,
           enc_W0, enc_b0, enc_W1, enc_b1,
           conv0_Wc, conv0_Ws, conv0_b, conv1_Wc, conv1_Ws, conv1_b,
           sc_W0, sc_b0, sc_W1, sc_b1, sc_W2, sc_b2, fin_W, fin_b):
    w0a, w0b = enc_W0[:20], enc_W0[20:]
    wc0 = conv0_Wc.sum(axis=0)
    pad_w = ((0, 0), (0, DH0 - 18))
    wc0a = jnp.pad(wc0[:, :18], pad_w)
    wc0b = jnp.pad(wc0[:, 18:], pad_w)
    wc1 = conv1_Wc.sum(axis=0)
    row = lambda b: b.reshape(1, -1)

    src = jnp.pad(a_res[0], (0, E_PAD - E)).reshape(E_PAD // CH, CH)
    dst = jnp.pad(a_res[1], (0, E_PAD - E), constant_values=N).reshape(
        E_PAD // CH, CH)

    (eo,) = _tc_call(
        _enc_body,
        [one_hot, features, w0a, w0b, row(enc_b0), enc_W1, row(enc_b1),
         wc0a, wc0b, conv0_Ws, row(conv0_b)],
        2, [128])

    p, _, _ = _sc_conv0()(eo, src, dst)

    (mo,) = _tc_call(
        _mid_body,
        [eo, p, wc1, conv1_Ws, row(conv1_b)],
        2, [128])

    q, _, _ = _sc_conv1()(mo, src, dst)

    (out,) = _tc_call(
        _score_body,
        [mo, q, sc_W0, row(sc_b0), sc_W1, row(sc_b1), sc_W2, row(sc_b2),
         fin_W, row(fin_b)],
        2, [1])

    return out[:N]
